# baseline (speedup 1.0000x reference)
import jax
import jax.numpy as jnp
from jax import lax
from jax.experimental import pallas as pl
from jax.experimental.pallas import tpu as pltpu
from jax.experimental.pallas import tpu_sc as plsc

B = 1024
L = 200
D = 128
N = B * L
NC = 2
NS = 16
NW = NC * NS
CHUNK = 128
NBUF = 5
LAG = 2

SLICES = [5, 15, 15, 15]
S = len(SLICES)


def _make_gather(nchunks):
    rows_per_w = nchunks * CHUNK
    ngroups = nchunks // NBUF

    def body(table_hbm, idx_hbm, out_hbm, idx_all, rows, idx_sem, g_sem, o_sem):
        wid = lax.axis_index("s") * NC + lax.axis_index("c")
        base = wid * rows_per_w

        pltpu.async_copy(idx_hbm.at[wid], idx_all, idx_sem).wait()

        def start_gather(ci, p):
            pltpu.async_copy(table_hbm.at[idx_all.at[ci]], rows.at[p], g_sem.at[p])

        def wait_gather(p):
            pltpu.make_async_copy(
                table_hbm.at[idx_all.at[0]], rows.at[p], g_sem.at[p]
            ).wait()

        def start_out(cj, q):
            off = base + cj * CHUNK
            pltpu.async_copy(rows.at[q], out_hbm.at[pl.ds(off, CHUNK)], o_sem.at[q])

        def wait_out(q):
            pltpu.make_async_copy(
                rows.at[q], out_hbm.at[pl.ds(0, CHUNK)], o_sem.at[q]
            ).wait()

        for b in range(NBUF):
            start_gather(b, b)
            cj = b - LAG
            if cj >= 0:
                q = cj % NBUF
                wait_gather(q)
                start_out(cj, q)

        def group(g, _):
            for b in range(NBUF):
                ci = g * NBUF + b
                wait_out(b)
                start_gather(ci, b)
                q = (b - LAG) % NBUF
                wait_gather(q)
                start_out(ci - LAG, q)
            return 0

        if ngroups > 1:
            lax.fori_loop(1, ngroups, group, 0)

        for k in range(LAG):
            ci = nchunks - LAG + k
            q = ci % NBUF
            wait_gather(q)
            start_out(ci, q)
        for q in range(NBUF):
            wait_out(q)

    return pl.kernel(
        body,
        out_type=jax.ShapeDtypeStruct((NW * rows_per_w, D), jnp.float32),
        mesh=plsc.VectorSubcoreMesh(
            core_axis_name="c", subcore_axis_name="s", num_cores=NC, num_subcores=NS
        ),
        scratch_types=[
            pltpu.VMEM((nchunks, CHUNK), jnp.int32),
            pltpu.VMEM((NBUF, CHUNK, D), jnp.float32),
            pltpu.SemaphoreType.DMA,
            pltpu.SemaphoreType.DMA((NBUF,)),
            pltpu.SemaphoreType.DMA((NBUF,)),
        ],
    )


_gathers = {n: _make_gather(n) for n in set(SLICES)}


BLK = 2048


def _mm_body(x_ref, w_ref, b_ref, o_ref):
    o_ref[...] = (
        jnp.dot(x_ref[...], w_ref[...], preferred_element_type=jnp.float32)
        + b_ref[...]
    )


def _mm_body_alias(x_ref, w_ref, b_ref, acc_ref, o_ref):
    del acc_ref
    o_ref[...] = (
        jnp.dot(x_ref[...], w_ref[...], preferred_element_type=jnp.float32)
        + b_ref[...]
    )


def _project_first(nrows, emb, W, b2d):
    return pl.pallas_call(
        _mm_body,
        grid=(nrows // BLK,),
        in_specs=[
            pl.BlockSpec((BLK, D), lambda i: (i, 0)),
            pl.BlockSpec((D, D), lambda i: (0, 0)),
            pl.BlockSpec((1, D), lambda i: (0, 0)),
        ],
        out_specs=pl.BlockSpec((BLK, D), lambda i: (i, 0)),
        out_shape=jax.ShapeDtypeStruct((N, D), jnp.float32),
    )(emb, W, b2d)


def _project_into(blk_off, nrows, emb, W, b2d, out_full):
    return pl.pallas_call(
        _mm_body_alias,
        grid=(nrows // BLK,),
        in_specs=[
            pl.BlockSpec((BLK, D), lambda i: (i, 0)),
            pl.BlockSpec((D, D), lambda i: (0, 0)),
            pl.BlockSpec((1, D), lambda i: (0, 0)),
            pl.BlockSpec(memory_space=pl.ANY),
        ],
        out_specs=pl.BlockSpec((BLK, D), lambda i, o=blk_off: (i + o, 0)),
        out_shape=jax.ShapeDtypeStruct((N, D), jnp.float32),
        input_output_aliases={3: 0},
    )(emb, W, b2d, out_full)


def kernel(input_ids, embed_table, W, b):
    idx = input_ids.reshape(N).astype(jnp.int32)
    b2d = b.reshape(1, D)

    embs = []
    row0 = 0
    for nchunks in SLICES:
        rows_s = NW * nchunks * CHUNK
        idx_s = lax.slice(idx, (row0,), (row0 + rows_s,)).reshape(NW, nchunks, CHUNK)
        embs.append(_gathers[nchunks](embed_table, idx_s))
        row0 += rows_s

    out = _project_first(NW * SLICES[0] * CHUNK, embs[0], W, b2d)
    row0 = NW * SLICES[0] * CHUNK
    for s in range(1, S):
        rows_s = NW * SLICES[s] * CHUNK
        out = _project_into(row0 // BLK, rows_s, embs[s], W, b2d, out)
        row0 += rows_s
    return out.reshape(B, L, D)


# speedup vs baseline: 1.1137x; 1.1137x over previous
import jax
import jax.numpy as jnp
from jax import lax
from jax.experimental import pallas as pl
from jax.experimental.pallas import tpu as pltpu
from jax.experimental.pallas import tpu_sc as plsc

B = 1024
L = 200
D = 128
N = B * L
NC = 2
NS = 16
NW = NC * NS
CHUNK = 128
NBUF = 5
LAG = 2

SLICES = [5, 15, 15, 15]
S = len(SLICES)


def _make_gather(nchunks):
    rows_per_w = nchunks * CHUNK
    ngroups = nchunks // NBUF

    def body(table_hbm, idx_hbm, out_hbm, idx_all, rows, idx_sem, g_sem, o_sem):
        wid = lax.axis_index("s") * NC + lax.axis_index("c")
        base = wid * rows_per_w

        pltpu.async_copy(idx_hbm.at[wid], idx_all, idx_sem).wait()

        def start_gather(ci, p):
            pltpu.async_copy(table_hbm.at[idx_all.at[ci]], rows.at[p], g_sem.at[p])

        def wait_gather(p):
            pltpu.make_async_copy(
                table_hbm.at[idx_all.at[0]], rows.at[p], g_sem.at[p]
            ).wait()

        def start_out(cj, q):
            off = base + cj * CHUNK
            pltpu.async_copy(rows.at[q], out_hbm.at[pl.ds(off, CHUNK)], o_sem.at[q])

        def wait_out(q):
            pltpu.make_async_copy(
                rows.at[q], out_hbm.at[pl.ds(0, CHUNK)], o_sem.at[q]
            ).wait()

        for b in range(NBUF):
            start_gather(b, b)
            cj = b - LAG
            if cj >= 0:
                q = cj % NBUF
                wait_gather(q)
                start_out(cj, q)

        def group(g, _):
            for b in range(NBUF):
                ci = g * NBUF + b
                wait_out(b)
                start_gather(ci, b)
                q = (b - LAG) % NBUF
                wait_gather(q)
                start_out(ci - LAG, q)
            return 0

        if ngroups > 1:
            lax.fori_loop(1, ngroups, group, 0)

        for k in range(LAG):
            ci = nchunks - LAG + k
            q = ci % NBUF
            wait_gather(q)
            start_out(ci, q)
        for q in range(NBUF):
            wait_out(q)

    return pl.kernel(
        body,
        out_type=jax.ShapeDtypeStruct((NW * rows_per_w, D), jnp.float32),
        mesh=plsc.VectorSubcoreMesh(
            core_axis_name="c", subcore_axis_name="s", num_cores=NC, num_subcores=NS
        ),
        scratch_types=[
            pltpu.VMEM((nchunks, CHUNK), jnp.int32),
            pltpu.VMEM((NBUF, CHUNK, D), jnp.float32),
            pltpu.SemaphoreType.DMA,
            pltpu.SemaphoreType.DMA((NBUF,)),
            pltpu.SemaphoreType.DMA((NBUF,)),
        ],
    )


_gathers = {n: _make_gather(n) for n in set(SLICES)}


BLK = 4096


def _mm_body(x_ref, w_ref, b_ref, o_ref):
    o_ref[...] = (
        jnp.dot(x_ref[...], w_ref[...], preferred_element_type=jnp.float32)
        + b_ref[...]
    )


def _mm_body_alias(x_ref, w_ref, b_ref, acc_ref, o_ref):
    del acc_ref
    o_ref[...] = (
        jnp.dot(x_ref[...], w_ref[...], preferred_element_type=jnp.float32)
        + b_ref[...]
    )


def _project_first(nrows, emb, W, b2d):
    return pl.pallas_call(
        _mm_body,
        grid=(nrows // BLK,),
        in_specs=[
            pl.BlockSpec((BLK, D), lambda i: (i, 0)),
            pl.BlockSpec((D, D), lambda i: (0, 0)),
            pl.BlockSpec((1, D), lambda i: (0, 0)),
        ],
        out_specs=pl.BlockSpec((BLK, D), lambda i: (i, 0)),
        out_shape=jax.ShapeDtypeStruct((N, D), jnp.float32),
    )(emb, W, b2d)


def _project_into(blk_off, nrows, emb, W, b2d, out_full):
    return pl.pallas_call(
        _mm_body_alias,
        grid=(nrows // BLK,),
        in_specs=[
            pl.BlockSpec((BLK, D), lambda i: (i, 0)),
            pl.BlockSpec((D, D), lambda i: (0, 0)),
            pl.BlockSpec((1, D), lambda i: (0, 0)),
            pl.BlockSpec(memory_space=pl.ANY),
        ],
        out_specs=pl.BlockSpec((BLK, D), lambda i, o=blk_off: (i + o, 0)),
        out_shape=jax.ShapeDtypeStruct((N, D), jnp.float32),
        input_output_aliases={3: 0},
    )(emb, W, b2d, out_full)


def kernel(input_ids, embed_table, W, b):
    idx = input_ids.reshape(N).astype(jnp.int32)
    b2d = b.reshape(1, D)

    embs = []
    row0 = 0
    for nchunks in SLICES:
        rows_s = NW * nchunks * CHUNK
        idx_s = lax.slice(idx, (row0,), (row0 + rows_s,)).reshape(NW, nchunks, CHUNK)
        embs.append(_gathers[nchunks](embed_table, idx_s))
        row0 += rows_s

    out = _project_first(NW * SLICES[0] * CHUNK, embs[0], W, b2d)
    row0 = NW * SLICES[0] * CHUNK
    for s in range(1, S):
        rows_s = NW * SLICES[s] * CHUNK
        out = _project_into(row0 // BLK, rows_s, embs[s], W, b2d, out)
        row0 += rows_s
    return out.reshape(B, L, D)


# speedup vs baseline: 1.1215x; 1.0070x over previous
import jax
import jax.numpy as jnp
from jax import lax
from jax.experimental import pallas as pl
from jax.experimental.pallas import tpu as pltpu
from jax.experimental.pallas import tpu_sc as plsc

B = 1024
L = 200
D = 128
N = B * L
NC = 2
NS = 16
NW = NC * NS
CHUNK = 128

SLICES = [2, 16, 16, 16]
S = len(SLICES)


def _ring_params(nchunks):
    for nbuf in (5, 4, 3, 2):
        if nchunks % nbuf == 0 and nchunks >= nbuf:
            return nbuf, min(2, nbuf - 1)
    return 1, 0


def _make_gather(nchunks):
    rows_per_w = nchunks * CHUNK
    NBUF, LAG = _ring_params(nchunks)
    ngroups = nchunks // NBUF

    def body(table_hbm, idx_hbm, out_hbm, idx_all, rows, idx_sem, g_sem, o_sem):
        wid = lax.axis_index("s") * NC + lax.axis_index("c")
        base = wid * rows_per_w

        pltpu.async_copy(idx_hbm.at[wid], idx_all, idx_sem).wait()

        def start_gather(ci, p):
            pltpu.async_copy(table_hbm.at[idx_all.at[ci]], rows.at[p], g_sem.at[p])

        def wait_gather(p):
            pltpu.make_async_copy(
                table_hbm.at[idx_all.at[0]], rows.at[p], g_sem.at[p]
            ).wait()

        def start_out(cj, q):
            off = base + cj * CHUNK
            pltpu.async_copy(rows.at[q], out_hbm.at[pl.ds(off, CHUNK)], o_sem.at[q])

        def wait_out(q):
            pltpu.make_async_copy(
                rows.at[q], out_hbm.at[pl.ds(0, CHUNK)], o_sem.at[q]
            ).wait()

        for b in range(NBUF):
            start_gather(b, b)
            cj = b - LAG
            if cj >= 0:
                q = cj % NBUF
                wait_gather(q)
                start_out(cj, q)

        def group(g, _):
            for b in range(NBUF):
                ci = g * NBUF + b
                wait_out(b)
                start_gather(ci, b)
                q = (b - LAG) % NBUF
                wait_gather(q)
                start_out(ci - LAG, q)
            return 0

        if ngroups > 1:
            lax.fori_loop(1, ngroups, group, 0)

        for k in range(LAG):
            ci = nchunks - LAG + k
            q = ci % NBUF
            wait_gather(q)
            start_out(ci, q)
        for q in range(NBUF):
            wait_out(q)

    return pl.kernel(
        body,
        out_type=jax.ShapeDtypeStruct((NW * rows_per_w, D), jnp.float32),
        mesh=plsc.VectorSubcoreMesh(
            core_axis_name="c", subcore_axis_name="s", num_cores=NC, num_subcores=NS
        ),
        scratch_types=[
            pltpu.VMEM((nchunks, CHUNK), jnp.int32),
            pltpu.VMEM((NBUF, CHUNK, D), jnp.float32),
            pltpu.SemaphoreType.DMA,
            pltpu.SemaphoreType.DMA((NBUF,)),
            pltpu.SemaphoreType.DMA((NBUF,)),
        ],
    )


_gathers = {n: _make_gather(n) for n in set(SLICES)}


BLK = 8192


def _mm_body(x_ref, w_ref, b_ref, o_ref):
    o_ref[...] = (
        jnp.dot(x_ref[...], w_ref[...], preferred_element_type=jnp.float32)
        + b_ref[...]
    )


def _mm_body_alias(x_ref, w_ref, b_ref, acc_ref, o_ref):
    del acc_ref
    o_ref[...] = (
        jnp.dot(x_ref[...], w_ref[...], preferred_element_type=jnp.float32)
        + b_ref[...]
    )


def _project_first(nrows, emb, W, b2d):
    return pl.pallas_call(
        _mm_body,
        grid=(nrows // BLK,),
        in_specs=[
            pl.BlockSpec((BLK, D), lambda i: (i, 0)),
            pl.BlockSpec((D, D), lambda i: (0, 0)),
            pl.BlockSpec((1, D), lambda i: (0, 0)),
        ],
        out_specs=pl.BlockSpec((BLK, D), lambda i: (i, 0)),
        out_shape=jax.ShapeDtypeStruct((N, D), jnp.float32),
    )(emb, W, b2d)


def _project_into(blk_off, nrows, emb, W, b2d, out_full):
    return pl.pallas_call(
        _mm_body_alias,
        grid=(nrows // BLK,),
        in_specs=[
            pl.BlockSpec((BLK, D), lambda i: (i, 0)),
            pl.BlockSpec((D, D), lambda i: (0, 0)),
            pl.BlockSpec((1, D), lambda i: (0, 0)),
            pl.BlockSpec(memory_space=pl.ANY),
        ],
        out_specs=pl.BlockSpec((BLK, D), lambda i, o=blk_off: (i + o, 0)),
        out_shape=jax.ShapeDtypeStruct((N, D), jnp.float32),
        input_output_aliases={3: 0},
    )(emb, W, b2d, out_full)


def kernel(input_ids, embed_table, W, b):
    idx = input_ids.reshape(N).astype(jnp.int32)
    b2d = b.reshape(1, D)

    embs = []
    row0 = 0
    for nchunks in SLICES:
        rows_s = NW * nchunks * CHUNK
        idx_s = lax.slice(idx, (row0,), (row0 + rows_s,)).reshape(NW, nchunks, CHUNK)
        embs.append(_gathers[nchunks](embed_table, idx_s))
        row0 += rows_s

    out = _project_first(NW * SLICES[0] * CHUNK, embs[0], W, b2d)
    row0 = NW * SLICES[0] * CHUNK
    for s in range(1, S):
        rows_s = NW * SLICES[s] * CHUNK
        out = _project_into(row0 // BLK, rows_s, embs[s], W, b2d, out)
        row0 += rows_s
    return out.reshape(B, L, D)


# speedup vs baseline: 1.1316x; 1.0090x over previous
import jax
import jax.numpy as jnp
from jax import lax
from jax.experimental import pallas as pl
from jax.experimental.pallas import tpu as pltpu
from jax.experimental.pallas import tpu_sc as plsc

B = 1024
L = 200
D = 128
N = B * L
NC = 2
NS = 16
NW = NC * NS
CHUNK = 128
NBUF = 5
LAG = 2

S = 5
NCHUNKS = N // (S * NW * CHUNK)
ROWS_PER_W = NCHUNKS * CHUNK
NSLICE = NW * ROWS_PER_W
NGROUPS = NCHUNKS // NBUF


def _make_gather(slice_base):
    def body(table_hbm, idx_hbm, out_hbm, idx_all, rows, idx_sem, g_sem, o_sem):
        wid = lax.axis_index("s") * NC + lax.axis_index("c")
        base = wid * ROWS_PER_W

        pltpu.async_copy(
            idx_hbm.at[pl.ds(slice_base + base, ROWS_PER_W)], idx_all, idx_sem
        ).wait()

        def start_gather(ci, p):
            pltpu.async_copy(
                table_hbm.at[idx_all.at[pl.ds(ci * CHUNK, CHUNK)]],
                rows.at[p],
                g_sem.at[p],
            )

        def wait_gather(p):
            pltpu.make_async_copy(
                table_hbm.at[idx_all.at[pl.ds(0, CHUNK)]], rows.at[p], g_sem.at[p]
            ).wait()

        def start_out(cj, q):
            off = base + cj * CHUNK
            pltpu.async_copy(rows.at[q], out_hbm.at[pl.ds(off, CHUNK)], o_sem.at[q])

        def wait_out(q):
            pltpu.make_async_copy(
                rows.at[q], out_hbm.at[pl.ds(0, CHUNK)], o_sem.at[q]
            ).wait()

        for b in range(NBUF):
            start_gather(b, b)
            cj = b - LAG
            if cj >= 0:
                q = cj % NBUF
                wait_gather(q)
                start_out(cj, q)

        def group(g, _):
            for b in range(NBUF):
                ci = g * NBUF + b
                wait_out(b)
                start_gather(ci, b)
                q = (b - LAG) % NBUF
                wait_gather(q)
                start_out(ci - LAG, q)
            return 0

        if NGROUPS > 1:
            lax.fori_loop(1, NGROUPS, group, 0)

        for k in range(LAG):
            ci = NCHUNKS - LAG + k
            q = ci % NBUF
            wait_gather(q)
            start_out(ci, q)
        for q in range(NBUF):
            wait_out(q)

    return pl.kernel(
        body,
        out_type=jax.ShapeDtypeStruct((NSLICE, D), jnp.float32),
        mesh=plsc.VectorSubcoreMesh(
            core_axis_name="c", subcore_axis_name="s", num_cores=NC, num_subcores=NS
        ),
        scratch_types=[
            pltpu.VMEM((ROWS_PER_W,), jnp.int32),
            pltpu.VMEM((NBUF, CHUNK, D), jnp.float32),
            pltpu.SemaphoreType.DMA,
            pltpu.SemaphoreType.DMA((NBUF,)),
            pltpu.SemaphoreType.DMA((NBUF,)),
        ],
    )


_gathers = [_make_gather(s * NSLICE) for s in range(S)]

BLK = 8192
BLKS_PER_SLICE = NSLICE // BLK


def _mm_body(x_ref, w_ref, b_ref, o_ref):
    o_ref[...] = (
        jnp.dot(x_ref[...], w_ref[...], preferred_element_type=jnp.float32)
        + b_ref[...]
    )


def _mm_body_alias(x_ref, w_ref, b_ref, acc_ref, o_ref):
    del acc_ref
    o_ref[...] = (
        jnp.dot(x_ref[...], w_ref[...], preferred_element_type=jnp.float32)
        + b_ref[...]
    )


def _project_first(emb, W, b2d):
    return pl.pallas_call(
        _mm_body,
        grid=(BLKS_PER_SLICE,),
        in_specs=[
            pl.BlockSpec((BLK, D), lambda i: (i, 0)),
            pl.BlockSpec((D, D), lambda i: (0, 0)),
            pl.BlockSpec((1, D), lambda i: (0, 0)),
        ],
        out_specs=pl.BlockSpec((BLK, D), lambda i: (i, 0)),
        out_shape=jax.ShapeDtypeStruct((N, D), jnp.float32),
    )(emb, W, b2d)


def _project_into(s, emb, W, b2d, out_full):
    return pl.pallas_call(
        _mm_body_alias,
        grid=(BLKS_PER_SLICE,),
        in_specs=[
            pl.BlockSpec((BLK, D), lambda i: (i, 0)),
            pl.BlockSpec((D, D), lambda i: (0, 0)),
            pl.BlockSpec((1, D), lambda i: (0, 0)),
            pl.BlockSpec(memory_space=pl.ANY),
        ],
        out_specs=pl.BlockSpec(
            (BLK, D), lambda i, s=s: (i + s * BLKS_PER_SLICE, 0)
        ),
        out_shape=jax.ShapeDtypeStruct((N, D), jnp.float32),
        input_output_aliases={3: 0},
    )(emb, W, b2d, out_full)


def kernel(input_ids, embed_table, W, b):
    idx = input_ids.reshape(N).astype(jnp.int32)
    b2d = b.reshape(1, D)

    embs = [_gathers[s](embed_table, idx) for s in range(S)]
    out = _project_first(embs[0], W, b2d)
    for s in range(1, S):
        out = _project_into(s, embs[s], W, b2d, out)
    return out.reshape(B, L, D)
